# Initial kernel scaffold; baseline (speedup 1.0000x reference)
import jax
import jax.numpy as jnp
from jax.experimental import pallas as pl


def kernel(u_pos, v_pos, v_neg, batch_size, u_emb, v_emb):
    raise NotImplementedError("write your pallas kernel here")


# speedup vs baseline: 1.6231x; 1.6231x over previous
import functools

import jax
import jax.numpy as jnp
from jax import lax
from jax.experimental import pallas as pl
from jax.experimental.pallas import tpu as pltpu
from jax.experimental.pallas import tpu_sc as plsc

L = 16


def _sc_scores(u_emb, v_emb, uidx, vidx, B, NW, NCHUNK, CH, S, D):
    BW = B // NW
    mesh = plsc.VectorSubcoreMesh(core_axis_name="c", subcore_axis_name="s")
    NC = 2

    @functools.partial(
        pl.kernel,
        mesh=mesh,
        compiler_params=pltpu.CompilerParams(
            needs_layout_passes=False, use_tc_tiling_on_sc=False),
        out_type=[
            jax.ShapeDtypeStruct((B,), jnp.float32),
            jax.ShapeDtypeStruct((B,), jnp.float32),
        ],
        scratch_types=[
            pltpu.VMEM((NCHUNK, CH), jnp.int32),
            pltpu.VMEM((S, NCHUNK, CH), jnp.int32),
            pltpu.VMEM((CH, D), jnp.float32),
            pltpu.VMEM((S, CH, D), jnp.float32),
            pltpu.VMEM((BW,), jnp.float32),
            pltpu.VMEM((BW,), jnp.float32),
            pltpu.SemaphoreType.DMA,
        ],
    )
    def sc_kernel(u_hbm, v_hbm, uidx_hbm, vidx_hbm, pos_hbm, neg_hbm,
                  uidx_v, vidx_v, ubuf, vbuf, pos_v, neg_v, sem):
        wid = lax.axis_index("s") * NC + lax.axis_index("c")
        pltpu.sync_copy(uidx_hbm.at[wid], uidx_v)
        pltpu.sync_copy(vidx_hbm.at[wid], vidx_v)
        lane = lax.broadcasted_iota(jnp.int32, (L,), 0)

        def do_chunk(c, carry):
            cps = [pltpu.async_copy(u_hbm.at[uidx_v.at[c]], ubuf, sem)]
            for s in range(S):
                cps.append(
                    pltpu.async_copy(v_hbm.at[vidx_v.at[s, c]], vbuf.at[s], sem))
            for cp in cps:
                cp.wait()
            for g in range(CH // L):
                rows = g * L + lane

                def dot_body(dd, acc):
                    ap, an = acc
                    col = jnp.broadcast_to(dd, (L,))
                    uu = plsc.load_gather(ubuf, [rows, col])
                    vv = plsc.load_gather(
                        vbuf, [jnp.full((L,), 0, jnp.int32), rows, col])
                    ns = plsc.load_gather(
                        vbuf, [jnp.full((L,), 1, jnp.int32), rows, col])
                    for s in range(2, S):
                        ns = ns + plsc.load_gather(
                            vbuf, [jnp.full((L,), s, jnp.int32), rows, col])
                    return ap + uu * vv, an + uu * ns

                zero = jnp.zeros((L,), jnp.float32)
                ap, an = lax.fori_loop(0, D, dot_body, (zero, zero))
                pos_v[pl.ds(c * CH + g * L, L)] = ap
                neg_v[pl.ds(c * CH + g * L, L)] = an
            return carry

        lax.fori_loop(0, NCHUNK, do_chunk, 0)
        base = wid * BW
        pltpu.sync_copy(pos_v, pos_hbm.at[pl.ds(base, BW)])
        pltpu.sync_copy(neg_v, neg_hbm.at[pl.ds(base, BW)])

    return sc_kernel(u_emb, v_emb, uidx, vidx)


def _tc_loss(pos, neg, bs):
    def body(bs_ref, p_ref, n_ref, o_ref):
        def logsig(t):
            return jnp.minimum(t, 0.0) - jnp.log1p(jnp.exp(-jnp.abs(t)))

        tot = jnp.sum(logsig(p_ref[...]) + logsig(-n_ref[...]))
        o_ref[0, 0] = -tot / bs_ref[0, 0]

    out = pl.pallas_call(
        body,
        out_shape=jax.ShapeDtypeStruct((1, 1), jnp.float32),
        in_specs=[
            pl.BlockSpec(memory_space=pltpu.SMEM),
            pl.BlockSpec(memory_space=pltpu.VMEM),
            pl.BlockSpec(memory_space=pltpu.VMEM),
        ],
        out_specs=pl.BlockSpec(memory_space=pltpu.SMEM),
    )(bs, pos, neg)
    return out[0, 0]


def kernel(u_pos, v_pos, v_neg, batch_size, u_emb, v_emb):
    B = u_pos.shape[0]
    NNEG = v_neg.shape[1]
    D = u_emb.shape[1]
    S = 1 + NNEG
    NW = 32
    BW = B // NW
    CH = 128
    NCHUNK = BW // CH

    uidx = u_pos.reshape(NW, NCHUNK, CH)
    vidx = jnp.concatenate([v_pos[None, :], v_neg.T], axis=0)
    vidx = vidx.reshape(S, NW, NCHUNK, CH).transpose(1, 0, 2, 3)

    pos, neg = _sc_scores(u_emb, v_emb, uidx, vidx, B, NW, NCHUNK, CH, S, D)

    r = B // 128
    bs = jnp.asarray(batch_size, jnp.float32).reshape(1, 1)
    return _tc_loss(pos.reshape(r, 128), neg.reshape(r, 128), bs)
